# Initial kernel scaffold; baseline (speedup 1.0000x reference)
import jax
import jax.numpy as jnp
from jax.experimental import pallas as pl


def kernel(x, W):
    raise NotImplementedError("write your pallas kernel here")


# speedup vs baseline: 1.1185x; 1.1185x over previous
import jax
import jax.numpy as jnp
from jax.experimental import pallas as pl
from jax.experimental.pallas import tpu as pltpu

_TOPK = 8
_E = 64
_D = 4096


def _router_block(x_ref, w_ref, scores_ref, ew_ref, ei_ref):
    x = x_ref[...]
    w = w_ref[...]
    logits = jax.lax.dot_general(
        x, w, (((1,), (1,)), ((), ())), preferred_element_type=jnp.float32
    )
    mx = jnp.max(logits, axis=1, keepdims=True)
    e = jnp.exp(logits - mx)
    denom = jnp.sum(e, axis=1, keepdims=True)
    scores = e / denom
    scores_ref[...] = scores

    iota = jax.lax.broadcasted_iota(jnp.int32, scores.shape, 1)
    s = scores
    vals, idxs = [], []
    for _ in range(_TOPK):
        m = jnp.max(s, axis=1, keepdims=True)
        idx = jnp.min(jnp.where(s == m, iota, _E), axis=1, keepdims=True)
        vals.append(m)
        idxs.append(idx)
        s = jnp.where(iota == idx, -1.0, s)
    w8 = jnp.concatenate(vals, axis=1)
    i8 = jnp.concatenate(idxs, axis=1)
    ew_ref[...] = w8 / jnp.sum(w8, axis=1, keepdims=True)
    ei_ref[...] = i8


def kernel(x, W):
    n_tokens = x.shape[0]
    block = 512
    grid = (n_tokens // block,)
    scores, ew, ei = pl.pallas_call(
        _router_block,
        grid=grid,
        in_specs=[
            pl.BlockSpec((block, _D), lambda i: (i, 0)),
            pl.BlockSpec((_E, _D), lambda i: (0, 0)),
        ],
        out_specs=[
            pl.BlockSpec((block, _E), lambda i: (i, 0)),
            pl.BlockSpec((block, _TOPK), lambda i: (i, 0)),
            pl.BlockSpec((block, _TOPK), lambda i: (i, 0)),
        ],
        out_shape=[
            jax.ShapeDtypeStruct((n_tokens, _E), jnp.float32),
            jax.ShapeDtypeStruct((n_tokens, _TOPK), jnp.float32),
            jax.ShapeDtypeStruct((n_tokens, _TOPK), jnp.int32),
        ],
    )(x, W)
    return (scores, ew, ei)


# speedup vs baseline: 1.2403x; 1.1089x over previous
import jax
import jax.numpy as jnp
from jax.experimental import pallas as pl
from jax.experimental.pallas import tpu as pltpu

_TOPK = 8
_E = 64
_D = 4096


def _router_block(x_ref, w_ref, scores_ref, ew_ref, ei_ref):
    x = x_ref[...]
    w = w_ref[...]
    logits = jax.lax.dot_general(
        x, w, (((1,), (1,)), ((), ())), preferred_element_type=jnp.float32
    )
    mx = jnp.max(logits, axis=1, keepdims=True)
    e = jnp.exp(logits - mx)
    denom = jnp.sum(e, axis=1, keepdims=True)
    scores = e / denom
    scores_ref[...] = scores

    iota = jax.lax.broadcasted_iota(jnp.int32, scores.shape, 1)
    s = scores
    vals, idxs = [], []
    for _ in range(_TOPK):
        m = jnp.max(s, axis=1, keepdims=True)
        idx = jnp.min(jnp.where(s == m, iota, _E), axis=1, keepdims=True)
        vals.append(m)
        idxs.append(idx)
        s = jnp.where(iota == idx, -1.0, s)
    w8 = jnp.concatenate(vals, axis=1)
    i8 = jnp.concatenate(idxs, axis=1)
    ew_ref[...] = w8 / jnp.sum(w8, axis=1, keepdims=True)
    ei_ref[...] = i8


def kernel(x, W):
    n_tokens = x.shape[0]
    block = 1024
    grid = (n_tokens // block,)
    scores, ew, ei = pl.pallas_call(
        _router_block,
        grid=grid,
        in_specs=[
            pl.BlockSpec((block, _D), lambda i: (i, 0)),
            pl.BlockSpec((_E, _D), lambda i: (0, 0)),
        ],
        out_specs=[
            pl.BlockSpec((block, _E), lambda i: (i, 0)),
            pl.BlockSpec((block, _TOPK), lambda i: (i, 0)),
            pl.BlockSpec((block, _TOPK), lambda i: (i, 0)),
        ],
        out_shape=[
            jax.ShapeDtypeStruct((n_tokens, _E), jnp.float32),
            jax.ShapeDtypeStruct((n_tokens, _TOPK), jnp.float32),
            jax.ShapeDtypeStruct((n_tokens, _TOPK), jnp.int32),
        ],
    )(x, W)
    return (scores, ew, ei)
